# baseline (speedup 1.0000x reference)
import functools

import jax
import jax.numpy as jnp
import numpy as np
from jax.experimental import pallas as pl

B = 16384
V = 27
JPAD = 32

_U32 = jnp.uint32
_K1 = np.uint32(0)
_K2 = np.uint32(42)
_K3 = np.uint32(0 ^ 42 ^ 0x1BD11BDA)
_TINY = np.float32(np.finfo(np.float32).tiny)


def _rotl(x, r):
    return (x << _U32(r)) | (x >> _U32(32 - r))


def _threefry_bits(n):
    rotations = ((13, 15, 26, 6), (17, 29, 16, 24))
    keyadds = ((_K2, _K3 + np.uint32(1)), (_K3, np.uint32(2)),
               (None, _K2 + np.uint32(3)), (_K2, _K3 + np.uint32(4)),
               (_K3, np.uint32(5)))
    x1 = n + _K2
    x0 = x1
    x1 = x0 ^ _rotl(x1, 13)
    for r in (15, 26, 6):
        x0 = x0 + x1
        x1 = x0 ^ _rotl(x1, r)
    x0 = x0 + keyadds[0][0]
    x1 = x1 + keyadds[0][1]
    for i in range(1, 5):
        for r in rotations[i % 2]:
            x0 = x0 + x1
            x1 = x0 ^ _rotl(x1, r)
        c0, c1 = keyadds[i]
        if c0 is not None:
            x0 = x0 + c0
        x1 = x1 + c1
    return x0 ^ x1


def _gumbel_from_bits(bits):
    fb = (bits >> _U32(9)) | _U32(0x3F800000)
    f = jax.lax.bitcast_convert_type(fb, jnp.float32) - jnp.float32(1.0)
    u = f * (jnp.float32(1.0) - _TINY) + _TINY
    u = jnp.maximum(_TINY, u)
    return -jnp.log(-jnp.log(u))


def _body(x_ref, lt_ref, out_ref):
    j = jax.lax.broadcasted_iota(jnp.int32, (JPAD, B), 0)
    i = jax.lax.broadcasted_iota(jnp.int32, (JPAD, B), 1)
    n = (i * V + j).astype(_U32)
    g = _gumbel_from_bits(_threefry_bits(n))

    tab = jnp.log(lt_ref[...])
    tab = jnp.concatenate([tab, jnp.full((V, JPAD - V), -1e30, jnp.float32)],
                          axis=1)
    onehot = (j == x_ref[...].reshape(1, B)).astype(jnp.float32)[:V]
    logp = jax.lax.dot_general(
        tab, onehot, (((0,), (0,)), ((), ())),
        preferred_element_type=jnp.float32,
        precision=jax.lax.Precision.HIGHEST)

    scores = g + logp

    val, idx = scores, j
    for size in (16, 8, 4, 2, 1):
        av, bv = val[:size], val[size:2 * size]
        ai, bi = idx[:size], idx[size:2 * size]
        takeb = (bv > av) | ((bv == av) & (bi < ai))
        val = jnp.where(takeb, bv, av)
        idx = jnp.where(takeb, bi, ai)
    out_ref[...] = idx


@functools.partial(jax.jit, static_argnames=())
def kernel(x, logits):
    out = pl.pallas_call(
        _body,
        out_shape=jax.ShapeDtypeStruct((1, B), jnp.int32),
    )(x.astype(jnp.int32), logits)
    return out.reshape(B, 1)


# speedup vs baseline: 1.0032x; 1.0032x over previous
import functools

import jax
import jax.numpy as jnp
import numpy as np
from jax.experimental import pallas as pl

B = 16384
V = 27
JPAD = 32

_U32 = jnp.uint32
_K1 = np.uint32(0)
_K2 = np.uint32(42)
_K3 = np.uint32(0 ^ 42 ^ 0x1BD11BDA)
_TINY = np.float32(np.finfo(np.float32).tiny)


def _rotl(x, r):
    return (x << _U32(r)) | (x >> _U32(32 - r))


def _threefry_bits(n):
    rotations = ((13, 15, 26, 6), (17, 29, 16, 24))
    keyadds = ((_K2, _K3 + np.uint32(1)), (_K3, np.uint32(2)),
               (None, _K2 + np.uint32(3)), (_K2, _K3 + np.uint32(4)),
               (_K3, np.uint32(5)))
    x1 = n + _K2
    x0 = x1
    x1 = x0 ^ _rotl(x1, 13)
    for r in (15, 26, 6):
        x0 = x0 + x1
        x1 = x0 ^ _rotl(x1, r)
    x0 = x0 + keyadds[0][0]
    x1 = x1 + keyadds[0][1]
    for i in range(1, 5):
        for r in rotations[i % 2]:
            x0 = x0 + x1
            x1 = x0 ^ _rotl(x1, r)
        c0, c1 = keyadds[i]
        if c0 is not None:
            x0 = x0 + c0
        x1 = x1 + c1
    return x0 ^ x1


def _gumbel_from_bits(bits):
    fb = (bits >> _U32(9)) | _U32(0x3F800000)
    f = jax.lax.bitcast_convert_type(fb, jnp.float32) - jnp.float32(1.0)
    u = f * (jnp.float32(1.0) - _TINY) + _TINY
    u = jnp.maximum(_TINY, u)
    return -jnp.log(-jnp.log(u))


def _body(x_ref, lt_ref, out_ref):
    j = jax.lax.broadcasted_iota(jnp.int32, (JPAD, B), 0)
    i = jax.lax.broadcasted_iota(jnp.int32, (JPAD, B), 1)
    n = (i * V + j).astype(_U32)
    g = _gumbel_from_bits(_threefry_bits(n))

    tab = jnp.log(lt_ref[...])
    tab = jnp.concatenate([tab, jnp.full((V, JPAD - V), -1e30, jnp.float32)],
                          axis=1)
    onehot = (j == x_ref[...].reshape(1, B)).astype(jnp.float32)[:V]
    logp = jax.lax.dot_general(
        tab, onehot, (((0,), (0,)), ((), ())),
        preferred_element_type=jnp.float32,
        precision=jax.lax.Precision.HIGHEST)

    scores = g + logp

    val, idx = scores, j
    for size in (16, 8, 4, 2, 1):
        av, bv = val[:size], val[size:2 * size]
        ai, bi = idx[:size], idx[size:2 * size]
        if size == 16:
            takeb = bv > av
        else:
            takeb = (bv > av) | ((bv == av) & (bi < ai))
        val = jnp.where(takeb, bv, av)
        idx = jnp.where(takeb, bi, ai)
    out_ref[...] = idx


@functools.partial(jax.jit, static_argnames=())
def kernel(x, logits):
    out = pl.pallas_call(
        _body,
        out_shape=jax.ShapeDtypeStruct((1, B), jnp.int32),
    )(x.astype(jnp.int32), logits)
    return out.reshape(B, 1)
